# baseline (speedup 1.0000x reference)
import functools

import jax
import jax.numpy as jnp
from jax import lax
from jax.experimental import pallas as pl
from jax.experimental.pallas import tpu as pltpu
from jax.experimental.pallas import tpu_sc as plsc

MAX_DEPTH = 20.0
VALID_THRESHOLD = 1e-06
MIN_VALID_POINTS = 10

B = 16
HW = 512 * 512
HALF = HW // 2
CHUNK = 32768
NCH = HALF // CHUNK
VPC = CHUNK // 16


def _sc_kernel(x_hbm, y_hbm, out_hbm, xv, yv, ov, stg, stg2, shared):
    c = lax.axis_index("c")
    s = lax.axis_index("s")
    batch = c * 8 + s // 2
    half = s % 2
    base = batch * HW + half * HALF

    zero16 = jnp.zeros((16,), jnp.float32)
    accs = (zero16, zero16, zero16, zero16, zero16)

    def chunk_body(k, accs):
        pltpu.sync_copy(x_hbm.at[pl.ds(base + k * CHUNK, CHUNK)], xv)
        pltpu.sync_copy(y_hbm.at[pl.ds(base + k * CHUNK, CHUNK)], yv)

        def body(t, accs):
            n_a, xs_a, xx_a, ys_a, xy_a = accs
            xr = xv[pl.ds(t * 16, 16)]
            yr = yv[pl.ds(t * 16, 16)]
            m = (yr > VALID_THRESHOLD) & (yr <= MAX_DEPTH)
            xm = jnp.where(m, xr, 0.0)
            ym = jnp.where(m, yr, 0.0)
            return (
                n_a + jnp.where(m, 1.0, 0.0),
                xs_a + xm,
                xx_a + xm * xm,
                ys_a + ym,
                xy_a + xm * ym,
            )

        return lax.fori_loop(0, VPC, body, accs)

    accs = lax.fori_loop(0, NCH, chunk_body, accs)

    for j, a in enumerate(accs):
        stg[j, :] = a
    pltpu.sync_copy(stg, shared.at[c, s])
    plsc.subcore_barrier()
    partner = s + 1 - 2 * (s % 2)
    pltpu.sync_copy(shared.at[c, partner], stg2)

    def lane_total(j):
        v = stg[j, :] + stg2[j, :]
        tot = v[0]
        for l in range(1, 16):
            tot = tot + v[l]
        return tot

    n = jnp.full((16,), lane_total(0))
    x_sum = jnp.full((16,), lane_total(1))
    x_sq_sum = jnp.full((16,), lane_total(2))
    y_sum = jnp.full((16,), lane_total(3))
    xy_sum = jnp.full((16,), lane_total(4))

    det = n * x_sq_sum - x_sum * x_sum
    valid = (n >= jnp.float32(MIN_VALID_POINTS)) & (jnp.abs(det) >= 1e-08)
    safe_det = jnp.where(valid, det, 1.0)
    scale = jnp.where(valid, (n * xy_sum - x_sum * y_sum) / safe_det, 1.0)
    bias = jnp.where(valid, (x_sq_sum * y_sum - x_sum * xy_sum) / safe_det, 0.0)

    def apply_chunk(k, carry):
        pltpu.sync_copy(x_hbm.at[pl.ds(base + k * CHUNK, CHUNK)], xv)

        def body(t, carry):
            ov[pl.ds(t * 16, 16)] = scale * xv[pl.ds(t * 16, 16)] + bias
            return carry

        lax.fori_loop(0, VPC, body, 0)
        pltpu.sync_copy(ov, out_hbm.at[pl.ds(base + k * CHUNK, CHUNK)])
        return carry

    lax.fori_loop(0, NCH, apply_chunk, 0)


def kernel(non_scale_dense, sparse_depth):
    b, ch, h, w = non_scale_dense.shape
    x = non_scale_dense.reshape(-1)
    y = sparse_depth.reshape(-1)
    run = functools.partial(
        pl.kernel,
        mesh=plsc.VectorSubcoreMesh(core_axis_name="c", subcore_axis_name="s"),
        out_type=jax.ShapeDtypeStruct((b * h * w,), jnp.float32),
        scratch_types=[
            pltpu.VMEM((CHUNK,), jnp.float32),
            pltpu.VMEM((CHUNK,), jnp.float32),
            pltpu.VMEM((CHUNK,), jnp.float32),
            pltpu.VMEM((5, 16), jnp.float32),
            pltpu.VMEM((5, 16), jnp.float32),
            pltpu.VMEM_SHARED((2, 16, 5, 16), jnp.float32),
        ],
    )(_sc_kernel)
    out = run(x, y)
    return out.reshape(b, ch, h, w)


# speedup vs baseline: 8.5841x; 8.5841x over previous
import jax
import jax.numpy as jnp
from jax.experimental import pallas as pl
from jax.experimental.pallas import tpu as pltpu

MAX_DEPTH = 20.0
VALID_THRESHOLD = 1e-06
MIN_VALID_POINTS = 10
BLOCK_B = 4


def _body(x_ref, y_ref, o_ref):
    h = x_ref.shape[1]
    x = x_ref[...]
    y = y_ref[...]
    xb = x.astype(jnp.bfloat16)
    yb = y.astype(jnp.bfloat16)
    mask = (yb > VALID_THRESHOLD) & (yb <= MAX_DEPTH)
    zero = jnp.bfloat16(0.0)
    xm = jnp.where(mask, xb, zero)
    ym = jnp.where(mask, yb, zero)
    xxm = xm * xm
    xym = xm * ym
    mf = jnp.where(mask, jnp.bfloat16(1.0), zero)
    ones = jnp.full((BLOCK_B, 8, h), 1.0, dtype=jnp.bfloat16)
    parts = [
        jax.lax.dot_general(
            ones, s, (((2,), (1,)), ((0,), (0,))),
            preferred_element_type=jnp.float32,
        )
        for s in (mf, xm, xxm, ym, xym)
    ]
    sums = jnp.sum(jnp.stack(parts), axis=(2, 3)) / 8.0
    n = sums[0]
    x_sum = sums[1]
    x_sq_sum = sums[2]
    y_sum = sums[3]
    xy_sum = sums[4]
    det = n * x_sq_sum - x_sum * x_sum
    valid = (n >= MIN_VALID_POINTS) & (jnp.abs(det) >= 1e-08)
    safe_det = jnp.where(valid, det, 1.0)
    scale = jnp.where(valid, (n * xy_sum - x_sum * y_sum) / safe_det, 1.0)
    bias = jnp.where(valid, (x_sq_sum * y_sum - x_sum * xy_sum) / safe_det, 0.0)
    o_ref[...] = scale[:, None, None] * x + bias[:, None, None]


def kernel(non_scale_dense, sparse_depth):
    b, c, h, w = non_scale_dense.shape
    x = non_scale_dense.reshape(b, h, w)
    y = sparse_depth.reshape(b, h, w)
    out = pl.pallas_call(
        _body,
        grid=(b // BLOCK_B,),
        in_specs=[
            pl.BlockSpec((BLOCK_B, h, w), lambda i: (i, 0, 0)),
            pl.BlockSpec((BLOCK_B, h, w), lambda i: (i, 0, 0)),
        ],
        out_specs=pl.BlockSpec((BLOCK_B, h, w), lambda i: (i, 0, 0)),
        out_shape=jax.ShapeDtypeStruct((b, h, w), x.dtype),
        compiler_params=pltpu.CompilerParams(
            dimension_semantics=("parallel",)),
    )(x, y)
    return out.reshape(b, c, h, w)


# speedup vs baseline: 9.3952x; 1.0945x over previous
import jax
import jax.numpy as jnp
from jax.experimental import pallas as pl
from jax.experimental.pallas import tpu as pltpu

MAX_DEPTH = 20.0
VALID_THRESHOLD = 1e-06
MIN_VALID_POINTS = 10

C = 2
NB = 8
NIN = 3
NOUT = 2


def _compute_chunk(xb, yb, ob, slot, oslot):
    h = xb.shape[2]
    ones = jnp.full((C, 8, h), 1.0, dtype=jnp.bfloat16)
    x = xb[slot]
    y = yb[slot]
    xf = x.astype(jnp.bfloat16)
    yf = y.astype(jnp.bfloat16)
    mask = (yf > VALID_THRESHOLD) & (yf <= MAX_DEPTH)
    zero = jnp.bfloat16(0.0)
    xm = jnp.where(mask, xf, zero)
    ym = jnp.where(mask, yf, zero)
    xxm = xm * xm
    xym = xm * ym
    mf = jnp.where(mask, jnp.bfloat16(1.0), zero)
    parts = [
        jax.lax.dot_general(
            ones, s, (((2,), (1,)), ((0,), (0,))),
            preferred_element_type=jnp.float32,
        )
        for s in (mf, xm, xxm, ym, xym)
    ]
    sums = jnp.sum(jnp.stack(parts), axis=(2, 3)) / 8.0
    n = sums[0]
    x_sum = sums[1]
    x_sq_sum = sums[2]
    y_sum = sums[3]
    xy_sum = sums[4]
    det = n * x_sq_sum - x_sum * x_sum
    valid = (n >= MIN_VALID_POINTS) & (jnp.abs(det) >= 1e-08)
    safe_det = jnp.where(valid, det, 1.0)
    scale = jnp.where(valid, (n * xy_sum - x_sum * y_sum) / safe_det, 1.0)
    bias = jnp.where(valid, (x_sq_sum * y_sum - x_sum * xy_sum) / safe_det, 0.0)
    ob[oslot] = scale[:, None, None] * x + bias[:, None, None]


def _body(x_hbm, y_hbm, o_hbm, xb, yb, ob, sx, sy, so):
    def in_copies(i):
        slot = i % NIN
        return (
            pltpu.make_async_copy(x_hbm.at[pl.ds(i * C, C)], xb.at[slot], sx.at[slot]),
            pltpu.make_async_copy(y_hbm.at[pl.ds(i * C, C)], yb.at[slot], sy.at[slot]),
        )

    def out_copy(i):
        oslot = i % NOUT
        return pltpu.make_async_copy(
            ob.at[oslot], o_hbm.at[pl.ds(i * C, C)], so.at[oslot])

    for i in range(NIN):
        for cp in in_copies(i):
            cp.start()
    for i in range(NB):
        for cp in in_copies(i):
            cp.wait()
        if i >= NOUT:
            out_copy(i - NOUT).wait()
        _compute_chunk(xb, yb, ob, i % NIN, i % NOUT)
        out_copy(i).start()
        if i + NIN < NB:
            for cp in in_copies(i + NIN):
                cp.start()
    out_copy(NB - 2).wait()
    out_copy(NB - 1).wait()


def kernel(non_scale_dense, sparse_depth):
    b, c, h, w = non_scale_dense.shape
    x = non_scale_dense.reshape(b, h, w)
    y = sparse_depth.reshape(b, h, w)
    out = pl.pallas_call(
        _body,
        in_specs=[
            pl.BlockSpec(memory_space=pltpu.HBM),
            pl.BlockSpec(memory_space=pltpu.HBM),
        ],
        out_specs=pl.BlockSpec(memory_space=pltpu.HBM),
        out_shape=jax.ShapeDtypeStruct((b, h, w), x.dtype),
        scratch_shapes=[
            pltpu.VMEM((NIN, C, h, w), jnp.float32),
            pltpu.VMEM((NIN, C, h, w), jnp.float32),
            pltpu.VMEM((NOUT, C, h, w), jnp.float32),
            pltpu.SemaphoreType.DMA((NIN,)),
            pltpu.SemaphoreType.DMA((NIN,)),
            pltpu.SemaphoreType.DMA((NOUT,)),
        ],
    )(x, y)
    return out.reshape(b, c, h, w)


# speedup vs baseline: 9.5063x; 1.0118x over previous
import jax
import jax.numpy as jnp
from jax.experimental import pallas as pl
from jax.experimental.pallas import tpu as pltpu

MAX_DEPTH = 20.0
VALID_THRESHOLD = 1e-06
MIN_VALID_POINTS = 10

C = 2
NB = 8
NIN = 4
NOUT = 2


def _compute_chunk(xb, yb, ob, slot, oslot):
    h = xb.shape[2]
    ones = jnp.full((C, 8, h), 1.0, dtype=jnp.bfloat16)
    x = xb[slot]
    y = yb[slot]
    xf = x.astype(jnp.bfloat16)
    yf = y.astype(jnp.bfloat16)
    mask = (yf > VALID_THRESHOLD) & (yf <= MAX_DEPTH)
    zero = jnp.bfloat16(0.0)
    xm = jnp.where(mask, xf, zero)
    ym = jnp.where(mask, yf, zero)
    xxm = xm * xm
    xym = xm * ym
    mf = jnp.where(mask, jnp.bfloat16(1.0), zero)
    parts = [
        jax.lax.dot_general(
            ones, s, (((2,), (1,)), ((0,), (0,))),
            preferred_element_type=jnp.float32,
        )
        for s in (mf, xm, xxm, ym, xym)
    ]
    sums = jnp.sum(jnp.stack(parts), axis=(2, 3)) / 8.0
    n = sums[0]
    x_sum = sums[1]
    x_sq_sum = sums[2]
    y_sum = sums[3]
    xy_sum = sums[4]
    det = n * x_sq_sum - x_sum * x_sum
    valid = (n >= MIN_VALID_POINTS) & (jnp.abs(det) >= 1e-08)
    safe_det = jnp.where(valid, det, 1.0)
    scale = jnp.where(valid, (n * xy_sum - x_sum * y_sum) / safe_det, 1.0)
    bias = jnp.where(valid, (x_sq_sum * y_sum - x_sum * xy_sum) / safe_det, 0.0)
    ob[oslot] = scale[:, None, None] * x + bias[:, None, None]


def _body(x_hbm, y_hbm, o_hbm, xb, yb, ob, sx, sy, so):
    def in_copies(i):
        slot = i % NIN
        return (
            pltpu.make_async_copy(x_hbm.at[pl.ds(i * C, C)], xb.at[slot], sx.at[slot]),
            pltpu.make_async_copy(y_hbm.at[pl.ds(i * C, C)], yb.at[slot], sy.at[slot]),
        )

    def out_copy(i):
        oslot = i % NOUT
        return pltpu.make_async_copy(
            ob.at[oslot], o_hbm.at[pl.ds(i * C, C)], so.at[oslot])

    for i in range(NIN):
        for cp in in_copies(i):
            cp.start()
    for i in range(NB):
        for cp in in_copies(i):
            cp.wait()
        if i >= NOUT:
            out_copy(i - NOUT).wait()
        _compute_chunk(xb, yb, ob, i % NIN, i % NOUT)
        out_copy(i).start()
        if i + NIN < NB:
            for cp in in_copies(i + NIN):
                cp.start()
    out_copy(NB - 2).wait()
    out_copy(NB - 1).wait()


def kernel(non_scale_dense, sparse_depth):
    b, c, h, w = non_scale_dense.shape
    x = non_scale_dense.reshape(b, h, w)
    y = sparse_depth.reshape(b, h, w)
    out = pl.pallas_call(
        _body,
        in_specs=[
            pl.BlockSpec(memory_space=pltpu.HBM),
            pl.BlockSpec(memory_space=pltpu.HBM),
        ],
        out_specs=pl.BlockSpec(memory_space=pltpu.HBM),
        out_shape=jax.ShapeDtypeStruct((b, h, w), x.dtype),
        scratch_shapes=[
            pltpu.VMEM((NIN, C, h, w), jnp.float32),
            pltpu.VMEM((NIN, C, h, w), jnp.float32),
            pltpu.VMEM((NOUT, C, h, w), jnp.float32),
            pltpu.SemaphoreType.DMA((NIN,)),
            pltpu.SemaphoreType.DMA((NIN,)),
            pltpu.SemaphoreType.DMA((NOUT,)),
        ],
    )(x, y)
    return out.reshape(b, c, h, w)


# speedup vs baseline: 9.5172x; 1.0012x over previous
import jax
import jax.numpy as jnp
from jax.experimental import pallas as pl
from jax.experimental.pallas import tpu as pltpu

MAX_DEPTH = 20.0
VALID_THRESHOLD = 1e-06
MIN_VALID_POINTS = 10

C = 1
NB = 16
NIN = 6
NOUT = 2


def _compute_chunk(xb, yb, ob, slot, oslot):
    h = xb.shape[2]
    ones = jnp.full((C, 8, h), 1.0, dtype=jnp.bfloat16)
    x = xb[slot]
    y = yb[slot]
    xf = x.astype(jnp.bfloat16)
    yf = y.astype(jnp.bfloat16)
    mask = (yf > VALID_THRESHOLD) & (yf <= MAX_DEPTH)
    zero = jnp.bfloat16(0.0)
    xm = jnp.where(mask, xf, zero)
    ym = jnp.where(mask, yf, zero)
    xxm = xm * xm
    xym = xm * ym
    mf = jnp.where(mask, jnp.bfloat16(1.0), zero)
    parts = [
        jax.lax.dot_general(
            ones, s, (((2,), (1,)), ((0,), (0,))),
            preferred_element_type=jnp.float32,
        )
        for s in (mf, xm, xxm, ym, xym)
    ]
    sums = jnp.sum(jnp.stack(parts), axis=(2, 3)) / 8.0
    n = sums[0]
    x_sum = sums[1]
    x_sq_sum = sums[2]
    y_sum = sums[3]
    xy_sum = sums[4]
    det = n * x_sq_sum - x_sum * x_sum
    valid = (n >= MIN_VALID_POINTS) & (jnp.abs(det) >= 1e-08)
    safe_det = jnp.where(valid, det, 1.0)
    scale = jnp.where(valid, (n * xy_sum - x_sum * y_sum) / safe_det, 1.0)
    bias = jnp.where(valid, (x_sq_sum * y_sum - x_sum * xy_sum) / safe_det, 0.0)
    ob[oslot] = scale[:, None, None] * x + bias[:, None, None]


def _body(x_hbm, y_hbm, o_hbm, xb, yb, ob, sx, sy, so):
    def in_copies(i):
        slot = i % NIN
        return (
            pltpu.make_async_copy(x_hbm.at[pl.ds(i * C, C)], xb.at[slot], sx.at[slot]),
            pltpu.make_async_copy(y_hbm.at[pl.ds(i * C, C)], yb.at[slot], sy.at[slot]),
        )

    def out_copy(i):
        oslot = i % NOUT
        return pltpu.make_async_copy(
            ob.at[oslot], o_hbm.at[pl.ds(i * C, C)], so.at[oslot])

    for i in range(NIN):
        for cp in in_copies(i):
            cp.start()
    for i in range(NB):
        for cp in in_copies(i):
            cp.wait()
        if i >= NOUT:
            out_copy(i - NOUT).wait()
        _compute_chunk(xb, yb, ob, i % NIN, i % NOUT)
        out_copy(i).start()
        if i + NIN < NB:
            for cp in in_copies(i + NIN):
                cp.start()
    out_copy(NB - 2).wait()
    out_copy(NB - 1).wait()


def kernel(non_scale_dense, sparse_depth):
    b, c, h, w = non_scale_dense.shape
    x = non_scale_dense.reshape(b, h, w)
    y = sparse_depth.reshape(b, h, w)
    out = pl.pallas_call(
        _body,
        in_specs=[
            pl.BlockSpec(memory_space=pltpu.HBM),
            pl.BlockSpec(memory_space=pltpu.HBM),
        ],
        out_specs=pl.BlockSpec(memory_space=pltpu.HBM),
        out_shape=jax.ShapeDtypeStruct((b, h, w), x.dtype),
        scratch_shapes=[
            pltpu.VMEM((NIN, C, h, w), jnp.float32),
            pltpu.VMEM((NIN, C, h, w), jnp.float32),
            pltpu.VMEM((NOUT, C, h, w), jnp.float32),
            pltpu.SemaphoreType.DMA((NIN,)),
            pltpu.SemaphoreType.DMA((NIN,)),
            pltpu.SemaphoreType.DMA((NOUT,)),
        ],
    )(x, y)
    return out.reshape(b, c, h, w)
